# baseline (speedup 1.0000x reference)
import jax
import jax.numpy as jnp
from jax.experimental import pallas as pl
from jax.experimental.pallas import tpu as pltpu

_BLOCK = 1000


def _elu(x):
    return jnp.where(x > 0, x, jnp.exp(x) - 1.0)


def _fused_block(x_ref, ent_ref, m_ref,
                 wg1_ref, bg1_ref, wg2_ref, bg2_ref,
                 we1_ref, be1_ref, we2_ref, be2_ref,
                 out_ref):
    xb = x_ref[...].astype(jnp.bfloat16)
    h = jnp.dot(xb, wg1_ref[...], preferred_element_type=jnp.float32)
    h = _elu(h + bg1_ref[...])
    g = jnp.dot(h.astype(jnp.bfloat16), wg2_ref[...],
                preferred_element_type=jnp.float32) + bg2_ref[...]

    eb = ent_ref[...].astype(jnp.bfloat16)
    he = jnp.dot(eb, we1_ref[...], preferred_element_type=jnp.float32)
    he = _elu(he + be1_ref[...])
    e = jnp.dot(he.astype(jnp.bfloat16), we2_ref[...],
                preferred_element_type=jnp.float32) + be2_ref[...]

    m = m_ref[...] > 0.5
    out_ref[:, :128] = jnp.where(m, g, 0.0)
    out_ref[:, 128:] = jnp.where(m, e, 0.0)


def _run_graph(x, ent, is_face, wg1, bg1, wg2, bg2, we1, be1, we2, be2,
               interpret=False):
    n = x.shape[0]
    gf = x.shape[1] * x.shape[2] * x.shape[3]
    ef = ent.shape[1]
    xf = x.reshape(n, gf)
    mf = is_face.reshape(n, 1)
    b = _BLOCK if n % _BLOCK == 0 else n
    grid = n // b

    row = lambda i: (i, 0)
    rep = lambda i: (0, 0)
    out = pl.pallas_call(
        _fused_block,
        grid=(grid,),
        in_specs=[
            pl.BlockSpec((b, gf), row),
            pl.BlockSpec((b, ef), row),
            pl.BlockSpec((b, 1), row),
            pl.BlockSpec((gf, 128), rep),
            pl.BlockSpec((1, 128), rep),
            pl.BlockSpec((128, 128), rep),
            pl.BlockSpec((1, 128), rep),
            pl.BlockSpec((ef, 128), rep),
            pl.BlockSpec((1, 128), rep),
            pl.BlockSpec((128, 128), rep),
            pl.BlockSpec((1, 128), rep),
        ],
        out_specs=pl.BlockSpec((b, 256), row),
        out_shape=jax.ShapeDtypeStruct((n, 256), jnp.float32),
        compiler_params=pltpu.CompilerParams(
            dimension_semantics=("parallel",)),
        interpret=interpret,
    )(xf, ent, mf, wg1, bg1, wg2, bg2, we1, be1, we2, be2)
    return out


def kernel(x1, ent1, is_face1, x2, ent2, is_face2,
           Wg1, bg1, Wg2, bg2, We1, be1, We2, be2):
    wg1 = Wg1.astype(jnp.bfloat16)
    wg2 = Wg2.astype(jnp.bfloat16)
    we1 = We1.astype(jnp.bfloat16)
    we2 = We2.astype(jnp.bfloat16)
    bg1r = bg1.reshape(1, -1)
    bg2r = bg2.reshape(1, -1)
    be1r = be1.reshape(1, -1)
    be2r = be2.reshape(1, -1)
    o1 = _run_graph(x1, ent1, is_face1, wg1, bg1r, wg2, bg2r,
                    we1, be1r, we2, be2r)
    o2 = _run_graph(x2, ent2, is_face2, wg1, bg1r, wg2, bg2r,
                    we1, be1r, we2, be2r)
    return (o1, o2)


# speedup vs baseline: 2.8942x; 2.8942x over previous
import jax
import jax.numpy as jnp
from jax.experimental import pallas as pl
from jax.experimental.pallas import tpu as pltpu

_BN = 512


def _elu(x):
    return jnp.where(x > 0, x, jnp.exp(x) - 1.0)


def _fused_block(x_ref, ent_ref, m_ref,
                 w1p_ref, bg1_ref, w2a_ref,
                 we1_ref, be1_ref, we2a_ref,
                 out_ref, s_ref):
    zero_row = jnp.zeros((1, _BN), dtype=jnp.bfloat16)
    for k in range(100):
        i, j = divmod(k, 10)
        s_ref[8 * k:8 * k + 7, :] = x_ref[i, j].astype(jnp.bfloat16)
        s_ref[8 * k + 7:8 * k + 8, :] = zero_row

    m = jnp.where(m_ref[...].reshape(1, _BN) > 0.5, 1.0, 0.0)
    mb = m.astype(jnp.bfloat16)

    h = jax.lax.dot_general(w1p_ref[...], s_ref[...],
                            (((0,), (0,)), ((), ())),
                            preferred_element_type=jnp.float32)
    h = _elu(h + bg1_ref[...])
    ha = jnp.concatenate([(h * m).astype(jnp.bfloat16), mb], axis=0)
    g = jax.lax.dot_general(ha, w2a_ref[...],
                            (((0,), (0,)), ((), ())),
                            preferred_element_type=jnp.float32)
    out_ref[:, :128] = g

    he = jax.lax.dot_general(we1_ref[...], ent_ref[...].astype(jnp.bfloat16),
                             (((0,), (0,)), ((), ())),
                             preferred_element_type=jnp.float32)
    he = _elu(he + be1_ref[...])
    hea = jnp.concatenate([(he * m).astype(jnp.bfloat16), mb], axis=0)
    e = jax.lax.dot_general(hea, we2a_ref[...],
                            (((0,), (0,)), ((), ())),
                            preferred_element_type=jnp.float32)
    out_ref[:, 128:] = e


def _run_graph(x, ent, is_face, w1p, bg1c, w2a, we1, be1c, we2a,
               interpret=False):
    n = x.shape[0]
    xt = jnp.transpose(x, (1, 2, 3, 0))
    entt = ent.T
    grid = pl.cdiv(n, _BN)

    out = pl.pallas_call(
        _fused_block,
        grid=(grid,),
        in_specs=[
            pl.BlockSpec((10, 10, 7, _BN), lambda i: (0, 0, 0, i)),
            pl.BlockSpec((9, _BN), lambda i: (0, i)),
            pl.BlockSpec((_BN,), lambda i: (i,)),
            pl.BlockSpec((800, 128), lambda i: (0, 0)),
            pl.BlockSpec((128, 1), lambda i: (0, 0)),
            pl.BlockSpec((129, 128), lambda i: (0, 0)),
            pl.BlockSpec((9, 128), lambda i: (0, 0)),
            pl.BlockSpec((128, 1), lambda i: (0, 0)),
            pl.BlockSpec((129, 128), lambda i: (0, 0)),
        ],
        out_specs=pl.BlockSpec((_BN, 256), lambda i: (i, 0)),
        out_shape=jax.ShapeDtypeStruct((n, 256), jnp.float32),
        scratch_shapes=[pltpu.VMEM((800, _BN), jnp.bfloat16)],
        compiler_params=pltpu.CompilerParams(
            dimension_semantics=("arbitrary",)),
        interpret=interpret,
    )(xt, entt, is_face, w1p, bg1c, w2a, we1, be1c, we2a)
    return out


def _prep_weights(Wg1, bg1, Wg2, bg2, We1, be1, We2, be2):
    w1p = jnp.pad(Wg1.reshape(100, 7, 128),
                  ((0, 0), (0, 1), (0, 0))).reshape(800, 128)
    w1p = w1p.astype(jnp.bfloat16)
    w2a = jnp.concatenate([Wg2, bg2[None, :]], axis=0).astype(jnp.bfloat16)
    we1 = We1.astype(jnp.bfloat16)
    we2a = jnp.concatenate([We2, be2[None, :]], axis=0).astype(jnp.bfloat16)
    return w1p, bg1.reshape(128, 1), w2a, we1, be1.reshape(128, 1), we2a


def kernel(x1, ent1, is_face1, x2, ent2, is_face2,
           Wg1, bg1, Wg2, bg2, We1, be1, We2, be2):
    w1p, bg1c, w2a, we1, be1c, we2a = _prep_weights(
        Wg1, bg1, Wg2, bg2, We1, be1, We2, be2)
    o1 = _run_graph(x1, ent1, is_face1, w1p, bg1c, w2a, we1, be1c, we2a)
    o2 = _run_graph(x2, ent2, is_face2, w1p, bg1c, w2a, we1, be1c, we2a)
    return (o1, o2)


# speedup vs baseline: 4.2162x; 1.4568x over previous
import jax
import jax.numpy as jnp
from jax.experimental import pallas as pl
from jax.experimental.pallas import tpu as pltpu

_BN = 1024


def _elu(x):
    return jnp.where(x > 0, x, jnp.exp(x) - 1.0)


def _fused_block(x_ref, ent_ref, m_ref,
                 w1p_ref, bg1_ref, w2a_ref,
                 we1_ref, be1_ref, we2a_ref,
                 out_ref, s_ref):
    @pl.when(pl.program_id(0) == 0)
    def _init():
        s_ref[...] = jnp.zeros_like(s_ref)

    for k in range(100):
        i, j = divmod(k, 10)
        s_ref[8 * k:8 * k + 7, :] = x_ref[i, j].astype(jnp.bfloat16)

    m = jnp.where(m_ref[...].reshape(1, _BN) > 0.5, 1.0, 0.0)
    mb = m.astype(jnp.bfloat16)

    h = jax.lax.dot_general(w1p_ref[...], s_ref[...],
                            (((0,), (0,)), ((), ())),
                            preferred_element_type=jnp.float32)
    h = _elu(h + bg1_ref[...])
    ha = jnp.concatenate([(h * m).astype(jnp.bfloat16), mb], axis=0)
    g = jax.lax.dot_general(ha, w2a_ref[...],
                            (((0,), (0,)), ((), ())),
                            preferred_element_type=jnp.float32)
    out_ref[:, :128] = g

    he = jax.lax.dot_general(we1_ref[...], ent_ref[...].astype(jnp.bfloat16),
                             (((0,), (0,)), ((), ())),
                             preferred_element_type=jnp.float32)
    he = _elu(he + be1_ref[...])
    hea = jnp.concatenate([(he * m).astype(jnp.bfloat16), mb], axis=0)
    e = jax.lax.dot_general(hea, we2a_ref[...],
                            (((0,), (0,)), ((), ())),
                            preferred_element_type=jnp.float32)
    out_ref[:, 128:] = e


def _run_graph(x, ent, is_face, w1p, bg1c, w2a, we1, be1c, we2a,
               interpret=False):
    n = x.shape[0]
    xt = jnp.transpose(x, (1, 2, 3, 0))
    entt = ent.T
    grid = pl.cdiv(n, _BN)

    out = pl.pallas_call(
        _fused_block,
        grid=(grid,),
        in_specs=[
            pl.BlockSpec((10, 10, 7, _BN), lambda i: (0, 0, 0, i)),
            pl.BlockSpec((9, _BN), lambda i: (0, i)),
            pl.BlockSpec((_BN,), lambda i: (i,)),
            pl.BlockSpec((800, 128), lambda i: (0, 0)),
            pl.BlockSpec((128, 1), lambda i: (0, 0)),
            pl.BlockSpec((129, 128), lambda i: (0, 0)),
            pl.BlockSpec((9, 128), lambda i: (0, 0)),
            pl.BlockSpec((128, 1), lambda i: (0, 0)),
            pl.BlockSpec((129, 128), lambda i: (0, 0)),
        ],
        out_specs=pl.BlockSpec((_BN, 256), lambda i: (i, 0)),
        out_shape=jax.ShapeDtypeStruct((n, 256), jnp.float32),
        scratch_shapes=[pltpu.VMEM((800, _BN), jnp.bfloat16)],
        compiler_params=pltpu.CompilerParams(
            dimension_semantics=("arbitrary",)),
        interpret=interpret,
    )(xt, entt, is_face, w1p, bg1c, w2a, we1, be1c, we2a)
    return out


def _prep_weights(Wg1, bg1, Wg2, bg2, We1, be1, We2, be2):
    w1p = jnp.pad(Wg1.reshape(100, 7, 128),
                  ((0, 0), (0, 1), (0, 0))).reshape(800, 128)
    w1p = w1p.astype(jnp.bfloat16)
    w2a = jnp.concatenate([Wg2, bg2[None, :]], axis=0).astype(jnp.bfloat16)
    we1 = We1.astype(jnp.bfloat16)
    we2a = jnp.concatenate([We2, be2[None, :]], axis=0).astype(jnp.bfloat16)
    return w1p, bg1.reshape(128, 1), w2a, we1, be1.reshape(128, 1), we2a


def kernel(x1, ent1, is_face1, x2, ent2, is_face2,
           Wg1, bg1, Wg2, bg2, We1, be1, We2, be2):
    w1p, bg1c, w2a, we1, be1c, we2a = _prep_weights(
        Wg1, bg1, Wg2, bg2, We1, be1, We2, be2)
    o1 = _run_graph(x1, ent1, is_face1, w1p, bg1c, w2a, we1, be1c, we2a)
    o2 = _run_graph(x2, ent2, is_face2, w1p, bg1c, w2a, we1, be1c, we2a)
    return (o1, o2)


# speedup vs baseline: 5.0834x; 1.2057x over previous
import jax
import jax.numpy as jnp
from jax.experimental import pallas as pl
from jax.experimental.pallas import tpu as pltpu

_BN = 2048


def _elu(x):
    return jnp.where(x > 0, x, jnp.exp(x) - 1.0)


def _fused_block(x_ref, ent_ref, m_ref,
                 w1p_ref, bg1_ref, w2a_ref,
                 we1_ref, be1_ref, we2a_ref,
                 out_ref, s_ref):
    @pl.when(pl.program_id(0) == 0)
    def _init():
        s_ref[...] = jnp.zeros_like(s_ref)

    for k in range(100):
        i, j = divmod(k, 10)
        s_ref[8 * k:8 * k + 7, :] = x_ref[i, j].astype(jnp.bfloat16)

    m = jnp.where(m_ref[...].reshape(1, _BN) > 0.5, 1.0, 0.0)
    mb = m.astype(jnp.bfloat16)

    h = jax.lax.dot_general(w1p_ref[...], s_ref[...],
                            (((0,), (0,)), ((), ())),
                            preferred_element_type=jnp.float32)
    h = _elu(h + bg1_ref[...])
    ha = jnp.concatenate([(h * m).astype(jnp.bfloat16), mb], axis=0)
    g = jax.lax.dot_general(ha, w2a_ref[...],
                            (((0,), (0,)), ((), ())),
                            preferred_element_type=jnp.float32)
    out_ref[:, :128] = g

    he = jax.lax.dot_general(we1_ref[...], ent_ref[...].astype(jnp.bfloat16),
                             (((0,), (0,)), ((), ())),
                             preferred_element_type=jnp.float32)
    he = _elu(he + be1_ref[...])
    hea = jnp.concatenate([(he * m).astype(jnp.bfloat16), mb], axis=0)
    e = jax.lax.dot_general(hea, we2a_ref[...],
                            (((0,), (0,)), ((), ())),
                            preferred_element_type=jnp.float32)
    out_ref[:, 128:] = e


def _run_graph(x, ent, is_face, w1p, bg1c, w2a, we1, be1c, we2a,
               interpret=False):
    n = x.shape[0]
    xt = jnp.transpose(x, (1, 2, 3, 0))
    entt = ent.T
    grid = pl.cdiv(n, _BN)

    out = pl.pallas_call(
        _fused_block,
        grid=(grid,),
        in_specs=[
            pl.BlockSpec((10, 10, 7, _BN), lambda i: (0, 0, 0, i)),
            pl.BlockSpec((9, _BN), lambda i: (0, i)),
            pl.BlockSpec((_BN,), lambda i: (i,)),
            pl.BlockSpec((800, 128), lambda i: (0, 0)),
            pl.BlockSpec((128, 1), lambda i: (0, 0)),
            pl.BlockSpec((129, 128), lambda i: (0, 0)),
            pl.BlockSpec((9, 128), lambda i: (0, 0)),
            pl.BlockSpec((128, 1), lambda i: (0, 0)),
            pl.BlockSpec((129, 128), lambda i: (0, 0)),
        ],
        out_specs=pl.BlockSpec((_BN, 256), lambda i: (i, 0)),
        out_shape=jax.ShapeDtypeStruct((n, 256), jnp.float32),
        scratch_shapes=[pltpu.VMEM((800, _BN), jnp.bfloat16)],
        compiler_params=pltpu.CompilerParams(
            dimension_semantics=("arbitrary",)),
        interpret=interpret,
    )(xt, entt, is_face, w1p, bg1c, w2a, we1, be1c, we2a)
    return out


def _prep_weights(Wg1, bg1, Wg2, bg2, We1, be1, We2, be2):
    w1p = jnp.pad(Wg1.reshape(100, 7, 128),
                  ((0, 0), (0, 1), (0, 0))).reshape(800, 128)
    w1p = w1p.astype(jnp.bfloat16)
    w2a = jnp.concatenate([Wg2, bg2[None, :]], axis=0).astype(jnp.bfloat16)
    we1 = We1.astype(jnp.bfloat16)
    we2a = jnp.concatenate([We2, be2[None, :]], axis=0).astype(jnp.bfloat16)
    return w1p, bg1.reshape(128, 1), w2a, we1, be1.reshape(128, 1), we2a


def kernel(x1, ent1, is_face1, x2, ent2, is_face2,
           Wg1, bg1, Wg2, bg2, We1, be1, We2, be2):
    w1p, bg1c, w2a, we1, be1c, we2a = _prep_weights(
        Wg1, bg1, Wg2, bg2, We1, be1, We2, be2)
    o1 = _run_graph(x1, ent1, is_face1, w1p, bg1c, w2a, we1, be1c, we2a)
    o2 = _run_graph(x2, ent2, is_face2, w1p, bg1c, w2a, we1, be1c, we2a)
    return (o1, o2)


# speedup vs baseline: 5.1371x; 1.0106x over previous
import jax
import jax.numpy as jnp
from jax.experimental import pallas as pl
from jax.experimental.pallas import tpu as pltpu

_BN = 4096


def _elu(x):
    return jnp.where(x > 0, x, jnp.exp(x) - 1.0)


def _fused_block(x_ref, ent_ref, m_ref,
                 w1p_ref, bg1_ref, w2a_ref,
                 we1_ref, be1_ref, we2a_ref,
                 out_ref, s_ref):
    @pl.when(pl.program_id(0) == 0)
    def _init():
        s_ref[...] = jnp.zeros_like(s_ref)

    for k in range(100):
        i, j = divmod(k, 10)
        s_ref[8 * k:8 * k + 7, :] = x_ref[i, j].astype(jnp.bfloat16)

    m = jnp.where(m_ref[...].reshape(1, _BN) > 0.5, 1.0, 0.0)
    mb = m.astype(jnp.bfloat16)

    h = jax.lax.dot_general(w1p_ref[...], s_ref[...],
                            (((0,), (0,)), ((), ())),
                            preferred_element_type=jnp.float32)
    h = _elu(h + bg1_ref[...])
    ha = jnp.concatenate([(h * m).astype(jnp.bfloat16), mb], axis=0)
    g = jax.lax.dot_general(ha, w2a_ref[...],
                            (((0,), (0,)), ((), ())),
                            preferred_element_type=jnp.float32)
    out_ref[:, :128] = g

    he = jax.lax.dot_general(we1_ref[...], ent_ref[...].astype(jnp.bfloat16),
                             (((0,), (0,)), ((), ())),
                             preferred_element_type=jnp.float32)
    he = _elu(he + be1_ref[...])
    hea = jnp.concatenate([(he * m).astype(jnp.bfloat16), mb], axis=0)
    e = jax.lax.dot_general(hea, we2a_ref[...],
                            (((0,), (0,)), ((), ())),
                            preferred_element_type=jnp.float32)
    out_ref[:, 128:] = e


def _run_graph(x, ent, is_face, w1p, bg1c, w2a, we1, be1c, we2a,
               interpret=False):
    n = x.shape[0]
    xt = jnp.transpose(x, (1, 2, 3, 0))
    entt = ent.T
    grid = pl.cdiv(n, _BN)

    out = pl.pallas_call(
        _fused_block,
        grid=(grid,),
        in_specs=[
            pl.BlockSpec((10, 10, 7, _BN), lambda i: (0, 0, 0, i)),
            pl.BlockSpec((9, _BN), lambda i: (0, i)),
            pl.BlockSpec((_BN,), lambda i: (i,)),
            pl.BlockSpec((800, 128), lambda i: (0, 0)),
            pl.BlockSpec((128, 1), lambda i: (0, 0)),
            pl.BlockSpec((129, 128), lambda i: (0, 0)),
            pl.BlockSpec((9, 128), lambda i: (0, 0)),
            pl.BlockSpec((128, 1), lambda i: (0, 0)),
            pl.BlockSpec((129, 128), lambda i: (0, 0)),
        ],
        out_specs=pl.BlockSpec((_BN, 256), lambda i: (i, 0)),
        out_shape=jax.ShapeDtypeStruct((n, 256), jnp.float32),
        scratch_shapes=[pltpu.VMEM((800, _BN), jnp.bfloat16)],
        compiler_params=pltpu.CompilerParams(
            dimension_semantics=("arbitrary",)),
        interpret=interpret,
    )(xt, entt, is_face, w1p, bg1c, w2a, we1, be1c, we2a)
    return out


def _prep_weights(Wg1, bg1, Wg2, bg2, We1, be1, We2, be2):
    w1p = jnp.pad(Wg1.reshape(100, 7, 128),
                  ((0, 0), (0, 1), (0, 0))).reshape(800, 128)
    w1p = w1p.astype(jnp.bfloat16)
    w2a = jnp.concatenate([Wg2, bg2[None, :]], axis=0).astype(jnp.bfloat16)
    we1 = We1.astype(jnp.bfloat16)
    we2a = jnp.concatenate([We2, be2[None, :]], axis=0).astype(jnp.bfloat16)
    return w1p, bg1.reshape(128, 1), w2a, we1, be1.reshape(128, 1), we2a


def kernel(x1, ent1, is_face1, x2, ent2, is_face2,
           Wg1, bg1, Wg2, bg2, We1, be1, We2, be2):
    w1p, bg1c, w2a, we1, be1c, we2a = _prep_weights(
        Wg1, bg1, Wg2, bg2, We1, be1, We2, be2)
    o1 = _run_graph(x1, ent1, is_face1, w1p, bg1c, w2a, we1, be1c, we2a)
    o2 = _run_graph(x2, ent2, is_face2, w1p, bg1c, w2a, we1, be1c, we2a)
    return (o1, o2)


# speedup vs baseline: 5.1493x; 1.0024x over previous
import jax
import jax.numpy as jnp
from jax.experimental import pallas as pl
from jax.experimental.pallas import tpu as pltpu

_BN = 2048


def _elu(x):
    return jnp.where(x > 0, x, jnp.exp(x) - 1.0)


def _mlp_pair(x_ref, ent_ref, m_ref, out_ref, s_ref,
              w1p_ref, bg1_ref, w2a_ref, we1_ref, be1_ref, we2a_ref):
    for k in range(100):
        i, j = divmod(k, 10)
        s_ref[8 * k:8 * k + 7, :] = x_ref[i, j].astype(jnp.bfloat16)

    m = jnp.where(m_ref[...].reshape(1, _BN) > 0.5, 1.0, 0.0)
    mb = m.astype(jnp.bfloat16)

    h = jax.lax.dot_general(w1p_ref[...], s_ref[...],
                            (((0,), (0,)), ((), ())),
                            preferred_element_type=jnp.float32)
    h = _elu(h + bg1_ref[...])
    ha = jnp.concatenate([(h * m).astype(jnp.bfloat16), mb], axis=0)
    g = jax.lax.dot_general(ha, w2a_ref[...],
                            (((0,), (0,)), ((), ())),
                            preferred_element_type=jnp.float32)
    out_ref[:, :128] = g

    he = jax.lax.dot_general(we1_ref[...], ent_ref[...].astype(jnp.bfloat16),
                             (((0,), (0,)), ((), ())),
                             preferred_element_type=jnp.float32)
    he = _elu(he + be1_ref[...])
    hea = jnp.concatenate([(he * m).astype(jnp.bfloat16), mb], axis=0)
    e = jax.lax.dot_general(hea, we2a_ref[...],
                            (((0,), (0,)), ((), ())),
                            preferred_element_type=jnp.float32)
    out_ref[:, 128:] = e


def _fused_block(x1_ref, ent1_ref, m1_ref, x2_ref, ent2_ref, m2_ref,
                 w1p_ref, bg1_ref, w2a_ref, we1_ref, be1_ref, we2a_ref,
                 out1_ref, out2_ref, s_ref):
    gsel = pl.program_id(0)

    @pl.when(jnp.logical_and(gsel == 0, pl.program_id(1) == 0))
    def _init():
        s_ref[...] = jnp.zeros_like(s_ref)

    @pl.when(gsel == 0)
    def _graph1():
        _mlp_pair(x1_ref, ent1_ref, m1_ref, out1_ref, s_ref,
                  w1p_ref, bg1_ref, w2a_ref, we1_ref, be1_ref, we2a_ref)

    @pl.when(gsel == 1)
    def _graph2():
        _mlp_pair(x2_ref, ent2_ref, m2_ref, out2_ref, s_ref,
                  w1p_ref, bg1_ref, w2a_ref, we1_ref, be1_ref, we2a_ref)


def _run_both(x1, ent1, m1, x2, ent2, m2,
              w1p, bg1c, w2a, we1, be1c, we2a, interpret=False):
    n = x1.shape[0]
    x1t = jnp.transpose(x1, (1, 2, 3, 0))
    x2t = jnp.transpose(x2, (1, 2, 3, 0))
    ent1t = ent1.T
    ent2t = ent2.T
    g = pl.cdiv(n, _BN)
    last = g - 1

    def a1(gi, i):
        return jnp.where(gi == 0, i, last)

    def a2(gi, i):
        return jnp.where(gi == 1, i, 0)

    rep = lambda gi, i: (0, 0)
    out1, out2 = pl.pallas_call(
        _fused_block,
        grid=(2, g),
        in_specs=[
            pl.BlockSpec((10, 10, 7, _BN), lambda gi, i: (0, 0, 0, a1(gi, i))),
            pl.BlockSpec((9, _BN), lambda gi, i: (0, a1(gi, i))),
            pl.BlockSpec((_BN,), lambda gi, i: (a1(gi, i),)),
            pl.BlockSpec((10, 10, 7, _BN), lambda gi, i: (0, 0, 0, a2(gi, i))),
            pl.BlockSpec((9, _BN), lambda gi, i: (0, a2(gi, i))),
            pl.BlockSpec((_BN,), lambda gi, i: (a2(gi, i),)),
            pl.BlockSpec((800, 128), rep),
            pl.BlockSpec((128, 1), rep),
            pl.BlockSpec((129, 128), rep),
            pl.BlockSpec((9, 128), rep),
            pl.BlockSpec((128, 1), rep),
            pl.BlockSpec((129, 128), rep),
        ],
        out_specs=[
            pl.BlockSpec((_BN, 256), lambda gi, i: (a1(gi, i), 0)),
            pl.BlockSpec((_BN, 256), lambda gi, i: (a2(gi, i), 0)),
        ],
        out_shape=[
            jax.ShapeDtypeStruct((n, 256), jnp.float32),
            jax.ShapeDtypeStruct((n, 256), jnp.float32),
        ],
        scratch_shapes=[pltpu.VMEM((800, _BN), jnp.bfloat16)],
        compiler_params=pltpu.CompilerParams(
            dimension_semantics=("arbitrary", "arbitrary")),
        interpret=interpret,
    )(x1t, ent1t, m1, x2t, ent2t, m2, w1p, bg1c, w2a, we1, be1c, we2a)
    return out1, out2


def _prep_weights(Wg1, bg1, Wg2, bg2, We1, be1, We2, be2):
    w1p = jnp.pad(Wg1.reshape(100, 7, 128),
                  ((0, 0), (0, 1), (0, 0))).reshape(800, 128)
    w1p = w1p.astype(jnp.bfloat16)
    w2a = jnp.concatenate([Wg2, bg2[None, :]], axis=0).astype(jnp.bfloat16)
    we1 = We1.astype(jnp.bfloat16)
    we2a = jnp.concatenate([We2, be2[None, :]], axis=0).astype(jnp.bfloat16)
    return w1p, bg1.reshape(128, 1), w2a, we1, be1.reshape(128, 1), we2a


def kernel(x1, ent1, is_face1, x2, ent2, is_face2,
           Wg1, bg1, Wg2, bg2, We1, be1, We2, be2):
    w1p, bg1c, w2a, we1, be1c, we2a = _prep_weights(
        Wg1, bg1, Wg2, bg2, We1, be1, We2, be2)
    o1, o2 = _run_both(x1, ent1, is_face1, x2, ent2, is_face2,
                       w1p, bg1c, w2a, we1, be1c, we2a)
    return (o1, o2)
